# baseline (speedup 1.0000x reference)
import jax
import jax.numpy as jnp
from jax import lax
from jax.experimental import pallas as pl
from jax.experimental.pallas import tpu as pltpu
from jax.experimental.pallas import tpu_sc as plsc

NV = 100000
NC = 100000
NE = 3200000
NG = 256
PCOEF = 3.0

NSUB = 16
NW = 32
EPW = NE // NW
ECH = 4000
NCHUNK = EPW // ECH
VCH = 1000
NVCH = NV // VCH
NBUF = 6


def _sc_body(xv_hbm, adjp_hbm, adjn_hbm, out_hbm, *rest):
    cidx = list(rest[0:NBUF])
    vidx = list(rest[NBUF:2 * NBUF])
    na = list(rest[2 * NBUF:3 * NBUF])
    nb = list(rest[3 * NBUF:4 * NBUF])
    (cb, xvs, accn, accd, semL, semG, semS) = rest[4 * NBUF:]
    cid = lax.axis_index("c")
    sid = lax.axis_index("s")
    w = cid * NSUB + sid

    for k in range((NVCH + NSUB - 1) // NSUB):
        t = sid + NSUB * k

        @pl.when(t < NVCH)
        def _():
            off = t * VCH
            pltpu.sync_copy(xv_hbm.at[pl.ds(off, VCH)], cb)
            pltpu.sync_copy(cb, xvs.at[pl.ds(off, VCH)])

    @pl.loop(0, VCH, step=16)
    def _(i):
        cb[pl.ds(i, 16)] = jnp.zeros((16,), jnp.float32)

    for k in range((NVCH + NSUB - 1) // NSUB):
        t = sid + NSUB * k

        @pl.when(t < NVCH)
        def _():
            pltpu.sync_copy(cb, accn.at[pl.ds(t * VCH, VCH)])
            pltpu.sync_copy(cb, accd.at[pl.ds(t * VCH, VCH)])

    plsc.subcore_barrier()

    NCH2 = 2 * NCHUNK

    def _src(j):
        if j < NCHUNK:
            return adjp_hbm, True, j
        return adjn_hbm, False, j - NCHUNK

    descL = [None] * NCH2
    descG = [None] * NCH2
    descS = [None] * NCH2
    for j in range(NCH2 + 3):
        if j < NCH2:
            b = j % NBUF
            if j >= NBUF:
                descS[j - NBUF][0].wait()
                descS[j - NBUF][1].wait()
            adj, _, jj = _src(j)
            off = w * EPW + jj * ECH
            descL[j] = (
                pltpu.async_copy(adj.at[pl.ds(off, ECH)], cidx[b],
                                 semL.at[b]),
                pltpu.async_copy(adj.at[pl.ds(NE + off, ECH)], vidx[b],
                                 semL.at[b]),
            )
        if 0 <= j - 1 < NCH2:
            jc = j - 1
            b = jc % NBUF
            descL[jc][0].wait()
            descL[jc][1].wait()
            descG[jc] = pltpu.async_copy(xvs.at[vidx[b]], na[b], semG.at[b])
        if 0 <= j - 2 < NCH2:
            jc = j - 2
            b = jc % NBUF
            descG[jc].wait()
            _, pos, _ = _src(jc)
            nar, nbr = na[b], nb[b]

            @pl.loop(0, ECH, step=64)
            def _(i):
                for u in range(4):
                    s = pl.ds(i + 16 * u, 16)
                    x = nar[s]
                    t = x if pos else 1.0 - x
                    e = jnp.exp(PCOEF * t)
                    nbr[s] = e
                    nar[s] = t * e
        if 0 <= j - 3 < NCH2:
            jc = j - 3
            b = jc % NBUF
            descS[jc] = (
                pltpu.async_copy(na[b], accn.at[cidx[b]], semS.at[b],
                                 add=True),
                pltpu.async_copy(nb[b], accd.at[cidx[b]], semS.at[b],
                                 add=True),
            )
    for jc in range(NCH2 - NBUF, NCH2):
        descS[jc][0].wait()
        descS[jc][1].wait()

    plsc.subcore_barrier()

    NDCH = NC // ECH
    for k in range((2 * NDCH + NSUB - 1) // NSUB):
        t = sid + NSUB * k

        @pl.when(t < NDCH)
        def _():
            o = t * ECH
            pltpu.sync_copy(accn.at[pl.ds(o, ECH)], na[0])
            pltpu.sync_copy(na[0], out_hbm.at[pl.ds(2 * cid * NC + o, ECH)])

        @pl.when((t >= NDCH) & (t < 2 * NDCH))
        def _():
            o = (t - NDCH) * ECH
            pltpu.sync_copy(accd.at[pl.ds(o, ECH)], na[1])
            pltpu.sync_copy(na[1],
                            out_hbm.at[pl.ds((2 * cid + 1) * NC + o, ECH)])


def _sc_edge_phase(xvf, adj_pos, adj_neg):
    mesh = plsc.VectorSubcoreMesh(core_axis_name="c", subcore_axis_name="s")
    return pl.kernel(
        _sc_body,
        out_type=jax.ShapeDtypeStruct((4 * NC,), jnp.float32),
        mesh=mesh,
        scratch_types=(
            [pltpu.VMEM((ECH,), jnp.int32) for _ in range(2 * NBUF)] +
            [pltpu.VMEM((ECH,), jnp.float32) for _ in range(2 * NBUF)] +
            [pltpu.VMEM((VCH,), jnp.float32),
             pltpu.VMEM_SHARED((NV,), jnp.float32),
             pltpu.VMEM_SHARED((NC,), jnp.float32),
             pltpu.VMEM_SHARED((NC,), jnp.float32),
             pltpu.SemaphoreType.DMA((NBUF,)),
             pltpu.SemaphoreType.DMA((NBUF,)),
             pltpu.SemaphoreType.DMA((NBUF,))]
        ),
    )(xvf, adj_pos, adj_neg)


_FR = 50
_FC = NC // _FR


def _tc_final_body(parts_ref, gidx_ref, cc_ref, out_ref):
    iota16 = lax.broadcasted_iota(jnp.int32, (16, 1), 0)

    def step(k, carry):
        acc, pen = carry
        num = parts_ref[0, pl.ds(k, 1), :] + parts_ref[2, pl.ds(k, 1), :]
        dom = parts_ref[1, pl.ds(k, 1), :] + parts_ref[3, pl.ds(k, 1), :]
        sm = num / dom
        pen = pen + jnp.sum(jnp.maximum(10.0 * (sm - 0.45), 0.0))
        g = gidx_ref[pl.ds(k, 1), :]
        mhi = jnp.where((g >> 4) == iota16, sm, 0.0)
        olo = ((g & 15) == iota16).astype(jnp.float32)
        acc = acc + lax.dot_general(mhi, olo, (((1,), (1,)), ((), ())),
                                    preferred_element_type=jnp.float32)
        return acc, pen

    acc, pen_sum = lax.fori_loop(
        0, _FR, step, (jnp.zeros((16, 16), jnp.float32), jnp.float32(0.0)))
    pg = acc / cc_ref[...]
    loss = jnp.mean((pg - 1.0) ** 2)
    out_ref[...] = jnp.stack([loss, loss - pen_sum * 0.005]).reshape(1, 2)


def kernel(xv, adj_pos, adj_neg, clause_count, gr_idx_cls, is_train):
    xvf = xv.reshape(NV)
    sc_out = _sc_edge_phase(xvf, adj_pos.reshape(2 * NE),
                            adj_neg.reshape(2 * NE))
    parts = sc_out.reshape(4, _FR, _FC)
    gidx = gr_idx_cls.reshape(_FR, _FC)
    cc = clause_count.reshape(16, 16)
    r = pl.pallas_call(
        _tc_final_body,
        out_shape=jax.ShapeDtypeStruct((1, 2), jnp.float32),
    )(parts, gidx, cc)
    return jnp.where(is_train, r[0, 1], r[0, 0])


# speedup vs baseline: 1.1691x; 1.1691x over previous
import jax
import jax.numpy as jnp
from jax import lax
from jax.experimental import pallas as pl
from jax.experimental.pallas import tpu as pltpu
from jax.experimental.pallas import tpu_sc as plsc

NV = 100000
NC = 100000
NE = 3200000
NG = 256
PCOEF = 3.0

NSUB = 16
NW = 32
EPW = NE // NW
ECH = 4000
NCHUNK = EPW // ECH
NBUF = 5
VCH = 1000
NVCH = NV // VCH


def _sc_body(xv_hbm, adjp_hbm, adjn_hbm, out_hbm, *rest):
    cidx = list(rest[0:NBUF])
    vidx = list(rest[NBUF:2 * NBUF])
    na = list(rest[2 * NBUF:3 * NBUF])
    nb = list(rest[3 * NBUF:4 * NBUF])
    (xb, cb, ap, bp, an, bn, accn, accd, semL, semG, semS) = rest[4 * NBUF:]
    cid = lax.axis_index("c")
    sid = lax.axis_index("s")
    w = cid * NSUB + sid

    for k in range((NVCH + NSUB - 1) // NSUB):
        t = sid + NSUB * k

        @pl.when(t < NVCH)
        def _():
            off = t * VCH
            pltpu.sync_copy(xv_hbm.at[pl.ds(off, VCH)], xb)

            @pl.loop(0, VCH, step=16)
            def _(i):
                x = xb[pl.ds(i, 16)]
                cb[pl.ds(i, 16)] = x * jnp.exp(PCOEF * x)
            pltpu.sync_copy(cb, ap.at[pl.ds(off, VCH)])

            @pl.loop(0, VCH, step=16)
            def _(i):
                x = xb[pl.ds(i, 16)]
                cb[pl.ds(i, 16)] = jnp.exp(PCOEF * x)
            pltpu.sync_copy(cb, bp.at[pl.ds(off, VCH)])

            @pl.loop(0, VCH, step=16)
            def _(i):
                x = 1.0 - xb[pl.ds(i, 16)]
                cb[pl.ds(i, 16)] = x * jnp.exp(PCOEF * x)
            pltpu.sync_copy(cb, an.at[pl.ds(off, VCH)])

            @pl.loop(0, VCH, step=16)
            def _(i):
                x = 1.0 - xb[pl.ds(i, 16)]
                cb[pl.ds(i, 16)] = jnp.exp(PCOEF * x)
            pltpu.sync_copy(cb, bn.at[pl.ds(off, VCH)])

    @pl.loop(0, VCH, step=16)
    def _(i):
        cb[pl.ds(i, 16)] = jnp.zeros((16,), jnp.float32)

    for k in range((NVCH + NSUB - 1) // NSUB):
        t = sid + NSUB * k

        @pl.when(t < NVCH)
        def _():
            pltpu.sync_copy(cb, accn.at[pl.ds(t * VCH, VCH)])
            pltpu.sync_copy(cb, accd.at[pl.ds(t * VCH, VCH)])

    plsc.subcore_barrier()

    NCH2 = 2 * NCHUNK

    def _src(j):
        if j < NCHUNK:
            return adjp_hbm, ap, bp, j
        return adjn_hbm, an, bn, j - NCHUNK

    descL = [None] * NCH2
    descG = [None] * NCH2
    descS = [None] * NCH2
    for j in range(NCH2 + 2):
        if j < NCH2:
            b = j % NBUF
            if j >= NBUF:
                descS[j - NBUF][0].wait()
                descS[j - NBUF][1].wait()
            adj, _, _, jj = _src(j)
            off = w * EPW + jj * ECH
            descL[j] = (
                pltpu.async_copy(adj.at[pl.ds(off, ECH)], cidx[b],
                                 semL.at[b]),
                pltpu.async_copy(adj.at[pl.ds(NE + off, ECH)], vidx[b],
                                 semL.at[b]),
            )
        if 0 <= j - 1 < NCH2:
            jc = j - 1
            b = jc % NBUF
            descL[jc][0].wait()
            descL[jc][1].wait()
            _, ta, tb, _ = _src(jc)
            descG[jc] = (
                pltpu.async_copy(ta.at[vidx[b]], na[b], semG.at[b]),
                pltpu.async_copy(tb.at[vidx[b]], nb[b], semG.at[b]),
            )
        if 0 <= j - 2 < NCH2:
            jc = j - 2
            b = jc % NBUF
            descG[jc][0].wait()
            descG[jc][1].wait()
            descS[jc] = (
                pltpu.async_copy(na[b], accn.at[cidx[b]], semS.at[b],
                                 add=True),
                pltpu.async_copy(nb[b], accd.at[cidx[b]], semS.at[b],
                                 add=True),
            )
    for jc in range(NCH2 - NBUF, NCH2):
        descS[jc][0].wait()
        descS[jc][1].wait()

    plsc.subcore_barrier()

    NDCH = NC // ECH
    for k in range((2 * NDCH + NSUB - 1) // NSUB):
        t = sid + NSUB * k

        @pl.when(t < NDCH)
        def _():
            o = t * ECH
            pltpu.sync_copy(accn.at[pl.ds(o, ECH)], na[0])
            pltpu.sync_copy(na[0], out_hbm.at[pl.ds(2 * cid * NC + o, ECH)])

        @pl.when((t >= NDCH) & (t < 2 * NDCH))
        def _():
            o = (t - NDCH) * ECH
            pltpu.sync_copy(accd.at[pl.ds(o, ECH)], na[1])
            pltpu.sync_copy(na[1],
                            out_hbm.at[pl.ds((2 * cid + 1) * NC + o, ECH)])


def _sc_edge_phase(xvf, adj_pos, adj_neg):
    mesh = plsc.VectorSubcoreMesh(core_axis_name="c", subcore_axis_name="s")
    return pl.kernel(
        _sc_body,
        out_type=jax.ShapeDtypeStruct((4 * NC,), jnp.float32),
        mesh=mesh,
        scratch_types=(
            [pltpu.VMEM((ECH,), jnp.int32) for _ in range(2 * NBUF)] +
            [pltpu.VMEM((ECH,), jnp.float32) for _ in range(2 * NBUF)] +
            [pltpu.VMEM((VCH,), jnp.float32),
             pltpu.VMEM((VCH,), jnp.float32),
             pltpu.VMEM_SHARED((NV,), jnp.float32),
             pltpu.VMEM_SHARED((NV,), jnp.float32),
             pltpu.VMEM_SHARED((NV,), jnp.float32),
             pltpu.VMEM_SHARED((NV,), jnp.float32),
             pltpu.VMEM_SHARED((NC,), jnp.float32),
             pltpu.VMEM_SHARED((NC,), jnp.float32),
             pltpu.SemaphoreType.DMA((NBUF,)),
             pltpu.SemaphoreType.DMA((NBUF,)),
             pltpu.SemaphoreType.DMA((NBUF,))]
        ),
    )(xvf, adj_pos, adj_neg)


_FR = 50
_FC = NC // _FR


def _tc_final_body(parts_ref, gidx_ref, cc_ref, out_ref):
    iota16 = lax.broadcasted_iota(jnp.int32, (16, 1), 0)

    def step(k, carry):
        acc, pen = carry
        num = parts_ref[0, pl.ds(k, 1), :] + parts_ref[2, pl.ds(k, 1), :]
        dom = parts_ref[1, pl.ds(k, 1), :] + parts_ref[3, pl.ds(k, 1), :]
        sm = num / dom
        pen = pen + jnp.sum(jnp.maximum(10.0 * (sm - 0.45), 0.0))
        g = gidx_ref[pl.ds(k, 1), :]
        mhi = jnp.where((g >> 4) == iota16, sm, 0.0)
        olo = ((g & 15) == iota16).astype(jnp.float32)
        acc = acc + lax.dot_general(mhi, olo, (((1,), (1,)), ((), ())),
                                    preferred_element_type=jnp.float32)
        return acc, pen

    acc, pen_sum = lax.fori_loop(
        0, _FR, step, (jnp.zeros((16, 16), jnp.float32), jnp.float32(0.0)))
    pg = acc / cc_ref[...]
    loss = jnp.mean((pg - 1.0) ** 2)
    out_ref[...] = jnp.stack([loss, loss - pen_sum * 0.005]).reshape(1, 2)


def kernel(xv, adj_pos, adj_neg, clause_count, gr_idx_cls, is_train):
    xvf = xv.reshape(NV)
    sc_out = _sc_edge_phase(xvf, adj_pos.reshape(2 * NE),
                            adj_neg.reshape(2 * NE))
    parts = sc_out.reshape(4, _FR, _FC)
    gidx = gr_idx_cls.reshape(_FR, _FC)
    cc = clause_count.reshape(16, 16)
    r = pl.pallas_call(
        _tc_final_body,
        out_shape=jax.ShapeDtypeStruct((1, 2), jnp.float32),
    )(parts, gidx, cc)
    return jnp.where(is_train, r[0, 1], r[0, 0])


# speedup vs baseline: 1.1780x; 1.0076x over previous
import jax
import jax.numpy as jnp
from jax import lax
from jax.experimental import pallas as pl
from jax.experimental.pallas import tpu as pltpu
from jax.experimental.pallas import tpu_sc as plsc

NV = 100000
NC = 100000
NE = 3200000
NG = 256
PCOEF = 3.0

NSUB = 16
NW = 32
EPW = NE // NW
ECH = 4000
NCHUNK = EPW // ECH
NBUF = 5
VCH = 2000
NVCH = NV // VCH


def _sc_body(xv_hbm, adjp_hbm, adjn_hbm, out_hbm, *rest):
    cidx = list(rest[0:NBUF])
    vidx = list(rest[NBUF:2 * NBUF])
    na = list(rest[2 * NBUF:3 * NBUF])
    nb = list(rest[3 * NBUF:4 * NBUF])
    (xb, cb, ap, bp, an, bn, accn, accd, semL, semG, semS) = rest[4 * NBUF:]
    cid = lax.axis_index("c")
    sid = lax.axis_index("s")
    w = cid * NSUB + sid

    for k in range((NVCH + NSUB - 1) // NSUB):
        t = sid + NSUB * k

        @pl.when(t < NVCH)
        def _():
            off = t * VCH
            pltpu.sync_copy(xv_hbm.at[pl.ds(off, VCH)], xb)

            @pl.loop(0, VCH, step=16)
            def _(i):
                x = xb[pl.ds(i, 16)]
                cb[pl.ds(i, 16)] = x * jnp.exp(PCOEF * x)
            pltpu.sync_copy(cb, ap.at[pl.ds(off, VCH)])

            @pl.loop(0, VCH, step=16)
            def _(i):
                x = xb[pl.ds(i, 16)]
                cb[pl.ds(i, 16)] = jnp.exp(PCOEF * x)
            pltpu.sync_copy(cb, bp.at[pl.ds(off, VCH)])

            @pl.loop(0, VCH, step=16)
            def _(i):
                x = 1.0 - xb[pl.ds(i, 16)]
                cb[pl.ds(i, 16)] = x * jnp.exp(PCOEF * x)
            pltpu.sync_copy(cb, an.at[pl.ds(off, VCH)])

            @pl.loop(0, VCH, step=16)
            def _(i):
                x = 1.0 - xb[pl.ds(i, 16)]
                cb[pl.ds(i, 16)] = jnp.exp(PCOEF * x)
            pltpu.sync_copy(cb, bn.at[pl.ds(off, VCH)])

    @pl.loop(0, VCH, step=16)
    def _(i):
        cb[pl.ds(i, 16)] = jnp.zeros((16,), jnp.float32)

    for k in range((NVCH + NSUB - 1) // NSUB):
        t = sid + NSUB * k

        @pl.when(t < NVCH)
        def _():
            pltpu.sync_copy(cb, accn.at[pl.ds(t * VCH, VCH)])
            pltpu.sync_copy(cb, accd.at[pl.ds(t * VCH, VCH)])

    plsc.subcore_barrier()

    NCH2 = 2 * NCHUNK

    def _src(j):
        if j < NCHUNK:
            return adjp_hbm, ap, bp, j
        return adjn_hbm, an, bn, j - NCHUNK

    descL = [None] * NCH2
    descG = [None] * NCH2
    descS = [None] * NCH2
    for j in range(NCH2 + 2):
        if j < NCH2:
            b = j % NBUF
            if j >= NBUF:
                descS[j - NBUF][0].wait()
                descS[j - NBUF][1].wait()
            adj, _, _, jj = _src(j)
            off = w * EPW + jj * ECH
            descL[j] = (
                pltpu.async_copy(adj.at[pl.ds(off, ECH)], cidx[b],
                                 semL.at[b]),
                pltpu.async_copy(adj.at[pl.ds(NE + off, ECH)], vidx[b],
                                 semL.at[b]),
            )
        if 0 <= j - 1 < NCH2:
            jc = j - 1
            b = jc % NBUF
            descL[jc][0].wait()
            descL[jc][1].wait()
            _, ta, tb, _ = _src(jc)
            descG[jc] = (
                pltpu.async_copy(ta.at[vidx[b]], na[b], semG.at[b]),
                pltpu.async_copy(tb.at[vidx[b]], nb[b], semG.at[b]),
            )
        if 0 <= j - 2 < NCH2:
            jc = j - 2
            b = jc % NBUF
            descG[jc][0].wait()
            descG[jc][1].wait()
            descS[jc] = (
                pltpu.async_copy(na[b], accn.at[cidx[b]], semS.at[b],
                                 add=True),
                pltpu.async_copy(nb[b], accd.at[cidx[b]], semS.at[b],
                                 add=True),
            )
    for jc in range(NCH2 - NBUF, NCH2):
        descS[jc][0].wait()
        descS[jc][1].wait()

    plsc.subcore_barrier()

    NDCH = NC // ECH
    for k in range((2 * NDCH + NSUB - 1) // NSUB):
        t = sid + NSUB * k

        @pl.when(t < NDCH)
        def _():
            o = t * ECH
            pltpu.sync_copy(accn.at[pl.ds(o, ECH)], na[0])
            pltpu.sync_copy(na[0], out_hbm.at[pl.ds(2 * cid * NC + o, ECH)])

        @pl.when((t >= NDCH) & (t < 2 * NDCH))
        def _():
            o = (t - NDCH) * ECH
            pltpu.sync_copy(accd.at[pl.ds(o, ECH)], na[1])
            pltpu.sync_copy(na[1],
                            out_hbm.at[pl.ds((2 * cid + 1) * NC + o, ECH)])


def _sc_edge_phase(xvf, adj_pos, adj_neg):
    mesh = plsc.VectorSubcoreMesh(core_axis_name="c", subcore_axis_name="s")
    return pl.kernel(
        _sc_body,
        out_type=jax.ShapeDtypeStruct((4 * NC,), jnp.float32),
        mesh=mesh,
        scratch_types=(
            [pltpu.VMEM((ECH,), jnp.int32) for _ in range(2 * NBUF)] +
            [pltpu.VMEM((ECH,), jnp.float32) for _ in range(2 * NBUF)] +
            [pltpu.VMEM((VCH,), jnp.float32),
             pltpu.VMEM((VCH,), jnp.float32),
             pltpu.VMEM_SHARED((NV,), jnp.float32),
             pltpu.VMEM_SHARED((NV,), jnp.float32),
             pltpu.VMEM_SHARED((NV,), jnp.float32),
             pltpu.VMEM_SHARED((NV,), jnp.float32),
             pltpu.VMEM_SHARED((NC,), jnp.float32),
             pltpu.VMEM_SHARED((NC,), jnp.float32),
             pltpu.SemaphoreType.DMA((NBUF,)),
             pltpu.SemaphoreType.DMA((NBUF,)),
             pltpu.SemaphoreType.DMA((NBUF,))]
        ),
    )(xvf, adj_pos, adj_neg)


_FR = 50
_FC = NC // _FR


def _tc_final_body(parts_ref, gidx_ref, cc_ref, out_ref):
    iota16 = lax.broadcasted_iota(jnp.int32, (16, 1), 0)

    def step(k, carry):
        acc, pen = carry
        num = parts_ref[0, pl.ds(k, 1), :] + parts_ref[2, pl.ds(k, 1), :]
        dom = parts_ref[1, pl.ds(k, 1), :] + parts_ref[3, pl.ds(k, 1), :]
        sm = num / dom
        pen = pen + jnp.sum(jnp.maximum(10.0 * (sm - 0.45), 0.0))
        g = gidx_ref[pl.ds(k, 1), :]
        mhi = jnp.where((g >> 4) == iota16, sm, 0.0)
        olo = ((g & 15) == iota16).astype(jnp.float32)
        acc = acc + lax.dot_general(mhi, olo, (((1,), (1,)), ((), ())),
                                    preferred_element_type=jnp.float32)
        return acc, pen

    acc, pen_sum = lax.fori_loop(
        0, _FR, step, (jnp.zeros((16, 16), jnp.float32), jnp.float32(0.0)))
    pg = acc / cc_ref[...]
    loss = jnp.mean((pg - 1.0) ** 2)
    out_ref[...] = jnp.stack([loss, loss - pen_sum * 0.005]).reshape(1, 2)


def kernel(xv, adj_pos, adj_neg, clause_count, gr_idx_cls, is_train):
    xvf = xv.reshape(NV)
    sc_out = _sc_edge_phase(xvf, adj_pos.reshape(2 * NE),
                            adj_neg.reshape(2 * NE))
    parts = sc_out.reshape(4, _FR, _FC)
    gidx = gr_idx_cls.reshape(_FR, _FC)
    cc = clause_count.reshape(16, 16)
    r = pl.pallas_call(
        _tc_final_body,
        out_shape=jax.ShapeDtypeStruct((1, 2), jnp.float32),
    )(parts, gidx, cc)
    return jnp.where(is_train, r[0, 1], r[0, 0])


# speedup vs baseline: 1.1844x; 1.0054x over previous
import jax
import jax.numpy as jnp
from jax import lax
from jax.experimental import pallas as pl
from jax.experimental.pallas import tpu as pltpu
from jax.experimental.pallas import tpu_sc as plsc

NV = 100000
NC = 100000
NE = 3200000
NG = 256
PCOEF = 3.0

NSUB = 16
NW = 32
EPW = NE // NW
ECH = 4000
NCHUNK = EPW // ECH
NBUF = 5
VCH = 4000
NVCH = NV // VCH


def _sc_body(xv_hbm, adjp_hbm, adjn_hbm, out_hbm, *rest):
    cidx = list(rest[0:NBUF])
    vidx = list(rest[NBUF:2 * NBUF])
    na = list(rest[2 * NBUF:3 * NBUF])
    nb = list(rest[3 * NBUF:4 * NBUF])
    (xb, cb, ap, bp, an, bn, accn, accd, semL, semG, semS) = rest[4 * NBUF:]
    cid = lax.axis_index("c")
    sid = lax.axis_index("s")
    w = cid * NSUB + sid

    for k in range((NVCH + NSUB - 1) // NSUB):
        t = sid + NSUB * k

        @pl.when(t < NVCH)
        def _():
            off = t * VCH
            pltpu.sync_copy(xv_hbm.at[pl.ds(off, VCH)], xb)

            @pl.loop(0, VCH, step=16)
            def _(i):
                x = xb[pl.ds(i, 16)]
                cb[pl.ds(i, 16)] = x * jnp.exp(PCOEF * x)
            pltpu.sync_copy(cb, ap.at[pl.ds(off, VCH)])

            @pl.loop(0, VCH, step=16)
            def _(i):
                x = xb[pl.ds(i, 16)]
                cb[pl.ds(i, 16)] = jnp.exp(PCOEF * x)
            pltpu.sync_copy(cb, bp.at[pl.ds(off, VCH)])

            @pl.loop(0, VCH, step=16)
            def _(i):
                x = 1.0 - xb[pl.ds(i, 16)]
                cb[pl.ds(i, 16)] = x * jnp.exp(PCOEF * x)
            pltpu.sync_copy(cb, an.at[pl.ds(off, VCH)])

            @pl.loop(0, VCH, step=16)
            def _(i):
                x = 1.0 - xb[pl.ds(i, 16)]
                cb[pl.ds(i, 16)] = jnp.exp(PCOEF * x)
            pltpu.sync_copy(cb, bn.at[pl.ds(off, VCH)])

    @pl.loop(0, VCH, step=16)
    def _(i):
        cb[pl.ds(i, 16)] = jnp.zeros((16,), jnp.float32)

    for k in range((NVCH + NSUB - 1) // NSUB):
        t = sid + NSUB * k

        @pl.when(t < NVCH)
        def _():
            pltpu.sync_copy(cb, accn.at[pl.ds(t * VCH, VCH)])
            pltpu.sync_copy(cb, accd.at[pl.ds(t * VCH, VCH)])

    plsc.subcore_barrier()

    NCH2 = 2 * NCHUNK

    def _src(j):
        if j < NCHUNK:
            return adjp_hbm, ap, bp, j
        return adjn_hbm, an, bn, j - NCHUNK

    descL = [None] * NCH2
    descG = [None] * NCH2
    descS = [None] * NCH2
    for j in range(NCH2 + 2):
        if j < NCH2:
            b = j % NBUF
            if j >= NBUF:
                descS[j - NBUF][0].wait()
                descS[j - NBUF][1].wait()
            adj, _, _, jj = _src(j)
            off = w * EPW + jj * ECH
            descL[j] = (
                pltpu.async_copy(adj.at[pl.ds(off, ECH)], cidx[b],
                                 semL.at[b]),
                pltpu.async_copy(adj.at[pl.ds(NE + off, ECH)], vidx[b],
                                 semL.at[b]),
            )
        if 0 <= j - 1 < NCH2:
            jc = j - 1
            b = jc % NBUF
            descL[jc][0].wait()
            descL[jc][1].wait()
            _, ta, tb, _ = _src(jc)
            descG[jc] = (
                pltpu.async_copy(ta.at[vidx[b]], na[b], semG.at[b]),
                pltpu.async_copy(tb.at[vidx[b]], nb[b], semG.at[b]),
            )
        if 0 <= j - 2 < NCH2:
            jc = j - 2
            b = jc % NBUF
            descG[jc][0].wait()
            descG[jc][1].wait()
            descS[jc] = (
                pltpu.async_copy(na[b], accn.at[cidx[b]], semS.at[b],
                                 add=True),
                pltpu.async_copy(nb[b], accd.at[cidx[b]], semS.at[b],
                                 add=True),
            )
    for jc in range(NCH2 - NBUF, NCH2):
        descS[jc][0].wait()
        descS[jc][1].wait()

    plsc.subcore_barrier()

    NDCH = NC // ECH
    for k in range((2 * NDCH + NSUB - 1) // NSUB):
        t = sid + NSUB * k

        @pl.when(t < NDCH)
        def _():
            o = t * ECH
            pltpu.sync_copy(accn.at[pl.ds(o, ECH)], na[0])
            pltpu.sync_copy(na[0], out_hbm.at[pl.ds(2 * cid * NC + o, ECH)])

        @pl.when((t >= NDCH) & (t < 2 * NDCH))
        def _():
            o = (t - NDCH) * ECH
            pltpu.sync_copy(accd.at[pl.ds(o, ECH)], na[1])
            pltpu.sync_copy(na[1],
                            out_hbm.at[pl.ds((2 * cid + 1) * NC + o, ECH)])


def _sc_edge_phase(xvf, adj_pos, adj_neg):
    mesh = plsc.VectorSubcoreMesh(core_axis_name="c", subcore_axis_name="s")
    return pl.kernel(
        _sc_body,
        out_type=jax.ShapeDtypeStruct((4 * NC,), jnp.float32),
        mesh=mesh,
        scratch_types=(
            [pltpu.VMEM((ECH,), jnp.int32) for _ in range(2 * NBUF)] +
            [pltpu.VMEM((ECH,), jnp.float32) for _ in range(2 * NBUF)] +
            [pltpu.VMEM((VCH,), jnp.float32),
             pltpu.VMEM((VCH,), jnp.float32),
             pltpu.VMEM_SHARED((NV,), jnp.float32),
             pltpu.VMEM_SHARED((NV,), jnp.float32),
             pltpu.VMEM_SHARED((NV,), jnp.float32),
             pltpu.VMEM_SHARED((NV,), jnp.float32),
             pltpu.VMEM_SHARED((NC,), jnp.float32),
             pltpu.VMEM_SHARED((NC,), jnp.float32),
             pltpu.SemaphoreType.DMA((NBUF,)),
             pltpu.SemaphoreType.DMA((NBUF,)),
             pltpu.SemaphoreType.DMA((NBUF,))]
        ),
    )(xvf, adj_pos, adj_neg)


_FR = 50
_FC = NC // _FR


def _tc_final_body(parts_ref, gidx_ref, cc_ref, out_ref):
    iota16 = lax.broadcasted_iota(jnp.int32, (16, 1), 0)

    def step(k, carry):
        acc, pen = carry
        num = parts_ref[0, pl.ds(k, 1), :] + parts_ref[2, pl.ds(k, 1), :]
        dom = parts_ref[1, pl.ds(k, 1), :] + parts_ref[3, pl.ds(k, 1), :]
        sm = num / dom
        pen = pen + jnp.sum(jnp.maximum(10.0 * (sm - 0.45), 0.0))
        g = gidx_ref[pl.ds(k, 1), :]
        mhi = jnp.where((g >> 4) == iota16, sm, 0.0)
        olo = ((g & 15) == iota16).astype(jnp.float32)
        acc = acc + lax.dot_general(mhi, olo, (((1,), (1,)), ((), ())),
                                    preferred_element_type=jnp.float32)
        return acc, pen

    acc, pen_sum = lax.fori_loop(
        0, _FR, step, (jnp.zeros((16, 16), jnp.float32), jnp.float32(0.0)))
    pg = acc / cc_ref[...]
    loss = jnp.mean((pg - 1.0) ** 2)
    out_ref[...] = jnp.stack([loss, loss - pen_sum * 0.005]).reshape(1, 2)


def kernel(xv, adj_pos, adj_neg, clause_count, gr_idx_cls, is_train):
    xvf = xv.reshape(NV)
    sc_out = _sc_edge_phase(xvf, adj_pos.reshape(2 * NE),
                            adj_neg.reshape(2 * NE))
    parts = sc_out.reshape(4, _FR, _FC)
    gidx = gr_idx_cls.reshape(_FR, _FC)
    cc = clause_count.reshape(16, 16)
    r = pl.pallas_call(
        _tc_final_body,
        out_shape=jax.ShapeDtypeStruct((1, 2), jnp.float32),
    )(parts, gidx, cc)
    return jnp.where(is_train, r[0, 1], r[0, 0])


# speedup vs baseline: 1.2017x; 1.0146x over previous
import jax
import jax.numpy as jnp
from jax import lax
from jax.experimental import pallas as pl
from jax.experimental.pallas import tpu as pltpu
from jax.experimental.pallas import tpu_sc as plsc

NV = 100000
NC = 100000
NE = 3200000
NG = 256
PCOEF = 3.0

NSUB = 16
NW = 32
EPW = NE // NW
ECH = 4000
NCHUNK = EPW // ECH
NBUF = 5
VCH = 4000
NVCH = NV // VCH


def _sc_body(xv_hbm, adjp_hbm, adjn_hbm, out_hbm, *rest):
    cidx = list(rest[0:NBUF])
    vidx = list(rest[NBUF:2 * NBUF])
    na = list(rest[2 * NBUF:3 * NBUF])
    nb = list(rest[3 * NBUF:4 * NBUF])
    (xb, cb, ap, bp, an, bn, accn, accd, semL, semG, semS) = rest[4 * NBUF:]
    cid = lax.axis_index("c")
    sid = lax.axis_index("s")
    w = cid * NSUB + sid

    for k in range((NVCH + NSUB - 1) // NSUB):
        t = sid + NSUB * k

        @pl.when(t < NVCH)
        def _():
            off = t * VCH
            pltpu.sync_copy(xv_hbm.at[pl.ds(off, VCH)], xb)

            @pl.loop(0, VCH, step=16)
            def _(i):
                s = pl.ds(i, 16)
                x = xb[s]
                e = jnp.exp(PCOEF * x)
                na[0][s] = x * e
                na[1][s] = e
                xn = 1.0 - x
                en = jnp.exp(PCOEF * xn)
                na[2][s] = xn * en
                na[3][s] = en
            pltpu.sync_copy(na[0], ap.at[pl.ds(off, VCH)])
            pltpu.sync_copy(na[1], bp.at[pl.ds(off, VCH)])
            pltpu.sync_copy(na[2], an.at[pl.ds(off, VCH)])
            pltpu.sync_copy(na[3], bn.at[pl.ds(off, VCH)])

    @pl.loop(0, VCH, step=16)
    def _(i):
        cb[pl.ds(i, 16)] = jnp.zeros((16,), jnp.float32)

    for k in range((NVCH + NSUB - 1) // NSUB):
        t = sid + NSUB * k

        @pl.when(t < NVCH)
        def _():
            pltpu.sync_copy(cb, accn.at[pl.ds(t * VCH, VCH)])
            pltpu.sync_copy(cb, accd.at[pl.ds(t * VCH, VCH)])

    plsc.subcore_barrier()

    NCH2 = 2 * NCHUNK

    def _src(j):
        if j < NCHUNK:
            return adjp_hbm, ap, bp, j
        return adjn_hbm, an, bn, j - NCHUNK

    descL = [None] * NCH2
    descG = [None] * NCH2
    descS = [None] * NCH2
    for j in range(NCH2 + 2):
        if j < NCH2:
            b = j % NBUF
            if j >= NBUF:
                descS[j - NBUF][0].wait()
                descS[j - NBUF][1].wait()
            adj, _, _, jj = _src(j)
            off = w * EPW + jj * ECH
            descL[j] = (
                pltpu.async_copy(adj.at[pl.ds(off, ECH)], cidx[b],
                                 semL.at[b]),
                pltpu.async_copy(adj.at[pl.ds(NE + off, ECH)], vidx[b],
                                 semL.at[b]),
            )
        if 0 <= j - 1 < NCH2:
            jc = j - 1
            b = jc % NBUF
            descL[jc][0].wait()
            descL[jc][1].wait()
            _, ta, tb, _ = _src(jc)
            descG[jc] = (
                pltpu.async_copy(ta.at[vidx[b]], na[b], semG.at[b]),
                pltpu.async_copy(tb.at[vidx[b]], nb[b], semG.at[b]),
            )
        if 0 <= j - 2 < NCH2:
            jc = j - 2
            b = jc % NBUF
            descG[jc][0].wait()
            descG[jc][1].wait()
            descS[jc] = (
                pltpu.async_copy(na[b], accn.at[cidx[b]], semS.at[b],
                                 add=True),
                pltpu.async_copy(nb[b], accd.at[cidx[b]], semS.at[b],
                                 add=True),
            )
    for jc in range(NCH2 - NBUF, NCH2):
        descS[jc][0].wait()
        descS[jc][1].wait()

    plsc.subcore_barrier()

    NDCH = NC // ECH
    for k in range((2 * NDCH + NSUB - 1) // NSUB):
        t = sid + NSUB * k

        @pl.when(t < NDCH)
        def _():
            o = t * ECH
            pltpu.sync_copy(accn.at[pl.ds(o, ECH)], na[0])
            pltpu.sync_copy(na[0], out_hbm.at[pl.ds(2 * cid * NC + o, ECH)])

        @pl.when((t >= NDCH) & (t < 2 * NDCH))
        def _():
            o = (t - NDCH) * ECH
            pltpu.sync_copy(accd.at[pl.ds(o, ECH)], na[1])
            pltpu.sync_copy(na[1],
                            out_hbm.at[pl.ds((2 * cid + 1) * NC + o, ECH)])


def _sc_edge_phase(xvf, adj_pos, adj_neg):
    mesh = plsc.VectorSubcoreMesh(core_axis_name="c", subcore_axis_name="s")
    return pl.kernel(
        _sc_body,
        out_type=jax.ShapeDtypeStruct((4 * NC,), jnp.float32),
        mesh=mesh,
        scratch_types=(
            [pltpu.VMEM((ECH,), jnp.int32) for _ in range(2 * NBUF)] +
            [pltpu.VMEM((ECH,), jnp.float32) for _ in range(2 * NBUF)] +
            [pltpu.VMEM((VCH,), jnp.float32),
             pltpu.VMEM((VCH,), jnp.float32),
             pltpu.VMEM_SHARED((NV,), jnp.float32),
             pltpu.VMEM_SHARED((NV,), jnp.float32),
             pltpu.VMEM_SHARED((NV,), jnp.float32),
             pltpu.VMEM_SHARED((NV,), jnp.float32),
             pltpu.VMEM_SHARED((NC,), jnp.float32),
             pltpu.VMEM_SHARED((NC,), jnp.float32),
             pltpu.SemaphoreType.DMA((NBUF,)),
             pltpu.SemaphoreType.DMA((NBUF,)),
             pltpu.SemaphoreType.DMA((NBUF,))]
        ),
    )(xvf, adj_pos, adj_neg)


_FR = 50
_FC = NC // _FR


def _tc_final_body(parts_ref, gidx_ref, cc_ref, out_ref):
    iota16 = lax.broadcasted_iota(jnp.int32, (16, 1), 0)

    def step(k, carry):
        acc, pen = carry
        num = parts_ref[0, pl.ds(k, 1), :] + parts_ref[2, pl.ds(k, 1), :]
        dom = parts_ref[1, pl.ds(k, 1), :] + parts_ref[3, pl.ds(k, 1), :]
        sm = num / dom
        pen = pen + jnp.sum(jnp.maximum(10.0 * (sm - 0.45), 0.0))
        g = gidx_ref[pl.ds(k, 1), :]
        mhi = jnp.where((g >> 4) == iota16, sm, 0.0)
        olo = ((g & 15) == iota16).astype(jnp.float32)
        acc = acc + lax.dot_general(mhi, olo, (((1,), (1,)), ((), ())),
                                    preferred_element_type=jnp.float32)
        return acc, pen

    acc, pen_sum = lax.fori_loop(
        0, _FR, step, (jnp.zeros((16, 16), jnp.float32), jnp.float32(0.0)))
    pg = acc / cc_ref[...]
    loss = jnp.mean((pg - 1.0) ** 2)
    out_ref[...] = jnp.stack([loss, loss - pen_sum * 0.005]).reshape(1, 2)


def kernel(xv, adj_pos, adj_neg, clause_count, gr_idx_cls, is_train):
    xvf = xv.reshape(NV)
    sc_out = _sc_edge_phase(xvf, adj_pos.reshape(2 * NE),
                            adj_neg.reshape(2 * NE))
    parts = sc_out.reshape(4, _FR, _FC)
    gidx = gr_idx_cls.reshape(_FR, _FC)
    cc = clause_count.reshape(16, 16)
    r = pl.pallas_call(
        _tc_final_body,
        out_shape=jax.ShapeDtypeStruct((1, 2), jnp.float32),
    )(parts, gidx, cc)
    return jnp.where(is_train, r[0, 1], r[0, 0])
